# baseline (speedup 1.0000x reference)
import functools

import jax
import jax.numpy as jnp
from jax import lax
from jax.experimental import pallas as pl
from jax.experimental.pallas import tpu as pltpu
from jax.experimental.pallas import tpu_sc as plsc

N_ANCH = 20000
N_CLS = 80
N_BINS = 16
TOP_K = 9
STRIDE = 8.0
MPAD = 128
BNM = 2000
NBLKM = N_ANCH // BNM
BIGF = 3.0e38
BIGI = 2**30


NP = 20480
GPT = 4


def _sc_thresh_kernel(a6_h, g6_h, out_h,
                      cxv, cyv, gtv, idxv0, idxv1, idxv2, idxv3,
                      av0, av1, av2, av3, bufv, sem):
    nc = 2
    wid = lax.axis_index("s") * nc + lax.axis_index("c")
    pltpu.sync_copy(a6_h.at[pl.ds(0, NP)], cxv)
    pltpu.sync_copy(a6_h.at[pl.ds(NP, NP)], cyv)
    pltpu.sync_copy(g6_h, gtv)
    lio = lax.iota(jnp.int32, 16)
    lio9 = lio < TOP_K
    _dn = lax.GatherDimensionNumbers(
        offset_dims=(), collapsed_slice_dims=(0,), start_index_map=(0,))

    def _shuf(v, pidx):
        return lax.gather(v, pidx, _dn, (1,),
                          mode=lax.GatherScatterMode.PROMISE_IN_BOUNDS)

    _pidx = [(lio ^ k).reshape(16, 1) for k in (8, 4, 2, 1)]

    def _splat_sum(v):
        for p in _pidx:
            v = v + _shuf(v, p)
        return v

    for g in range(GPT):
        gid = wid * GPT + g

        def _gt(row, gid=gid):
            return gtv[pl.ds(row * (MPAD * 16) + gid * 16, 16)]

        gcx = _gt(0)
        gcy = _gt(1)
        gx0 = _gt(2)
        gy0 = _gt(3)
        gx1 = _gt(4)
        gy1 = _gt(5)

        def chunk_body(c, carry):
            ba = list(carry[0:TOP_K])
            ia = list(carry[TOP_K:2 * TOP_K])
            bb = list(carry[2 * TOP_K:3 * TOP_K])
            ib = list(carry[3 * TOP_K:])
            for u in range(2):
                base_a = c * 64 + u * 32
                base_b = base_a + 16
                cxa = cxv[pl.ds(base_a, 16)]
                cya = cyv[pl.ds(base_a, 16)]
                cxb = cxv[pl.ds(base_b, 16)]
                cyb = cyv[pl.ds(base_b, 16)]
                dxa = cxa - gcx
                dya = cya - gcy
                dxb = cxb - gcx
                dyb = cyb - gcy
                ta = dxa * dxa + dya * dya
                tb_ = dxb * dxb + dyb * dyb
                tia = lio + base_a
                tib = lio + base_b
                for j in range(TOP_K):
                    la = ta < ba[j]
                    lb = tb_ < bb[j]
                    nka = jnp.minimum(ta, ba[j])
                    nkb = jnp.minimum(tb_, bb[j])
                    xka = jnp.maximum(ta, ba[j])
                    xkb = jnp.maximum(tb_, bb[j])
                    nia = jnp.where(la, tia, ia[j])
                    nib = jnp.where(lb, tib, ib[j])
                    tia = jnp.where(la, ia[j], tia)
                    tib = jnp.where(lb, ib[j], tib)
                    ta = xka
                    tb_ = xkb
                    ba[j] = nka
                    ia[j] = nia
                    bb[j] = nkb
                    ib[j] = nib
            return tuple(ba) + tuple(ia) + tuple(bb) + tuple(ib)

        init = (tuple(jnp.full((16,), BIGF, jnp.float32)
                      for _ in range(TOP_K))
                + tuple(jnp.full((16,), BIGI, jnp.int32)
                        for _ in range(TOP_K))) * 2
        carry = lax.fori_loop(0, NP // 64, chunk_body, init)
        bk = carry[0:TOP_K] + carry[2 * TOP_K:3 * TOP_K]
        bi = carry[TOP_K:2 * TOP_K] + carry[3 * TOP_K:]

        last_k = jnp.full((16,), -1.0, jnp.float32)
        last_i = jnp.full((16,), -1, jnp.int32)
        selvec = jnp.zeros((16,), jnp.int32)
        for r in range(TOP_K):
            ck = jnp.full((16,), BIGF, jnp.float32)
            ci = jnp.full((16,), BIGI, jnp.int32)
            for j in range(2 * TOP_K):
                valid = (bk[j] > last_k) | ((bk[j] == last_k)
                                            & (bi[j] > last_i))
                kj = jnp.where(valid, bk[j], BIGF)
                ij = jnp.where(valid, bi[j], BIGI)
                lt = (kj < ck) | ((kj == ck) & (ij < ci))
                ck = jnp.where(lt, kj, ck)
                ci = jnp.where(lt, ij, ci)
            for p in _pidx:
                pk = _shuf(ck, p)
                pi = _shuf(ci, p)
                lt = (pk < ck) | ((pk == ck) & (pi < ci))
                ck = jnp.where(lt, pk, ck)
                ci = jnp.where(lt, pi, ci)
            last_k = ck
            last_i = ci
            selvec = jnp.where(lio == r, ci, selvec)

        selm = jnp.where(lio9, selvec, 0)
        idxv0[...] = selm + 2 * NP
        idxv1[...] = selm + 3 * NP
        idxv2[...] = selm + 4 * NP
        idxv3[...] = selm + 5 * NP
        h0 = pltpu.async_copy(a6_h.at[idxv0], av0, sem)
        h1 = pltpu.async_copy(a6_h.at[idxv1], av1, sem)
        h2 = pltpu.async_copy(a6_h.at[idxv2], av2, sem)
        h3 = pltpu.async_copy(a6_h.at[idxv3], av3, sem)
        h0.wait()
        h1.wait()
        h2.wait()
        h3.wait()
        ax0 = av0[...]
        ay0 = av1[...]
        ax1 = av2[...]
        ay1 = av3[...]
        area_a = (ax1 - ax0) * (ay1 - ay0)
        area_g = (gx1 - gx0) * (gy1 - gy0)
        iw = jnp.maximum(jnp.minimum(ax1, gx1) - jnp.maximum(ax0, gx0), 0.0)
        ih = jnp.maximum(jnp.minimum(ay1, gy1) - jnp.maximum(ay0, gy0), 0.0)
        inter = iw * ih
        union = area_a + area_g - inter
        iou = inter / jnp.maximum(union, 1e-9)
        s1 = _splat_sum(jnp.where(lio9, iou, 0.0))
        mean = s1 / jnp.float32(TOP_K)
        dev = iou - mean
        var = _splat_sum(jnp.where(lio9, dev * dev, 0.0)) \
            / jnp.float32(TOP_K - 1)
        seed = lax.bitcast_convert_type(
            (lax.bitcast_convert_type(var, jnp.int32) >> 1) + 0x1FBD1DF5,
            jnp.float32)
        y = seed
        for _ in range(3):
            y = 0.5 * (y + var / y)
        stdv = jnp.where(var > 0.0, y, 0.0)
        bufv[g, :] = mean + stdv

    pltpu.sync_copy(bufv, out_h.at[pl.ds(wid * GPT, GPT)])


def _sc_thresh_call(a6, g6):
    mesh = plsc.VectorSubcoreMesh(core_axis_name="c", subcore_axis_name="s")
    fn = functools.partial(
        pl.kernel,
        out_type=jax.ShapeDtypeStruct((MPAD, 16), jnp.float32),
        mesh=mesh,
        scratch_types=[
            pltpu.VMEM((NP,), jnp.float32),
            pltpu.VMEM((NP,), jnp.float32),
            pltpu.VMEM((6 * MPAD * 16,), jnp.float32),
            pltpu.VMEM((16,), jnp.int32),
            pltpu.VMEM((16,), jnp.int32),
            pltpu.VMEM((16,), jnp.int32),
            pltpu.VMEM((16,), jnp.int32),
            pltpu.VMEM((16,), jnp.float32),
            pltpu.VMEM((16,), jnp.float32),
            pltpu.VMEM((16,), jnp.float32),
            pltpu.VMEM((16,), jnp.float32),
            pltpu.VMEM((GPT, 16), jnp.float32),
            pltpu.SemaphoreType.DMA,
        ],
    )(_sc_thresh_kernel)
    return fn(a6, g6)


def _main_kernel(a_ref, cls_ref, reg_ref, gt_ref, tbl_ref, gmask_ref,
                 th_ref, out_ref):
    i = pl.program_id(0)
    ax0 = a_ref[:, 0:1]
    ay0 = a_ref[:, 1:2]
    ax1 = a_ref[:, 2:3]
    ay1 = a_ref[:, 3:4]
    gx0 = gt_ref[0:1, :]
    gy0 = gt_ref[1:2, :]
    gx1 = gt_ref[2:3, :]
    gy1 = gt_ref[3:4, :]
    acx = (ax0 + ax1) * 0.5
    acy = (ay0 + ay1) * 0.5
    area_a = (ax1 - ax0) * (ay1 - ay0)
    area_g = (gx1 - gx0) * (gy1 - gy0)
    iw = jnp.maximum(jnp.minimum(ax1, gx1) - jnp.maximum(ax0, gx0), 0.0)
    ih = jnp.maximum(jnp.minimum(ay1, gy1) - jnp.maximum(ay0, gy0), 0.0)
    inter = iw * ih
    iou = inter / jnp.maximum(area_a + area_g - inter, 1e-9)
    thresh = th_ref[0:1, :]
    inside = ((acx >= gx0) & (acx <= gx1) & (acy >= gy0) & (acy <= gy1))
    pos = (iou >= thresh) & inside
    lanem = jax.lax.broadcasted_iota(jnp.int32, (BNM, MPAD), 1)
    matched = jnp.max(jnp.where(pos, lanem, -1), axis=1, keepdims=True)
    sel = lanem == matched
    self_f = sel.astype(jnp.float32)
    gath = jnp.dot(self_f, tbl_ref[...],
                   preferred_element_type=jnp.float32)
    maxiou = jnp.dot(self_f * iou, tbl_ref[...],
                     preferred_element_type=jnp.float32)[:, 5:6]
    tbx0 = gath[:, 0:1]
    tby0 = gath[:, 1:2]
    tbx1 = gath[:, 2:3]
    tby1 = gath[:, 3:4]
    label = gath[:, 4:5]
    posf = gath[:, 5:6]
    q = maxiou * posf

    p = cls_ref[...]
    cio = jax.lax.broadcasted_iota(
        jnp.int32, (BNM, N_CLS), 1).astype(jnp.float32)
    t = jnp.where(cio == label, q, 0.0)
    bce = (jnp.maximum(p, 0.0) - p * t
           + jnp.log(1.0 + jnp.exp(-jnp.abs(p))))
    bce_row = jnp.sum(bce, axis=0, keepdims=True)

    tl = (acx - tbx0) / STRIDE
    tt = (acy - tby0) / STRIDE
    tr = (tbx1 - acx) / STRIDE
    tb = (tby1 - acy) / STRIDE
    r64 = reg_ref[...]
    bio = jax.lax.broadcasted_iota(jnp.int32, (BNM, 4 * N_BINS), 1)
    gid = bio // N_BINS
    binf = (bio % N_BINS).astype(jnp.float32)
    tgt64 = jnp.where(gid == 0, tl,
                      jnp.where(gid == 1, tt,
                                jnp.where(gid == 2, tr, tb)))
    tgt64 = jnp.clip(tgt64, 0.0, N_BINS - 1 - 1e-3)
    w = jnp.maximum(1.0 - jnp.abs(binf - tgt64), 0.0)
    ex = jnp.exp(r64)
    gm = gmask_ref[...]
    s4 = jnp.dot(ex, gm, preferred_element_type=jnp.float32)[:, 0:4]
    t4 = jnp.dot(r64 * w, gm, preferred_element_type=jnp.float32)[:, 0:4]
    elem4 = (jnp.log(s4) - t4) * posf
    reg_row = jnp.sum(elem4, axis=0, keepdims=True)
    npos_row = jnp.sum(posf, axis=0, keepdims=True)

    @pl.when(i == 0)
    def _():
        out_ref[...] = jnp.zeros(out_ref.shape, jnp.float32)

    out_ref[0:1, 0:N_CLS] += bce_row
    out_ref[1:2, 0:4] += reg_row
    out_ref[2:3, 0:1] += npos_row


def _main_call(anchors, cls_preds, reg64, gt_t, tbl, gmask, thresh):
    return pl.pallas_call(
        _main_kernel,
        grid=(NBLKM,),
        in_specs=[
            pl.BlockSpec((BNM, 4), lambda b: (b, 0)),
            pl.BlockSpec((BNM, N_CLS), lambda b: (b, 0)),
            pl.BlockSpec((BNM, 4 * N_BINS), lambda b: (b, 0)),
            pl.BlockSpec((8, MPAD), lambda b: (0, 0)),
            pl.BlockSpec((MPAD, MPAD), lambda b: (0, 0)),
            pl.BlockSpec((4 * N_BINS, MPAD), lambda b: (0, 0)),
            pl.BlockSpec((8, MPAD), lambda b: (0, 0)),
        ],
        out_specs=pl.BlockSpec((8, MPAD), lambda b: (0, 0)),
        out_shape=jax.ShapeDtypeStruct((8, MPAD), jnp.float32),
        compiler_params=pltpu.CompilerParams(
            dimension_semantics=("arbitrary",)),
    )(anchors, cls_preds, reg64, gt_t, tbl, gmask, thresh)


def kernel(cls_preds, reg_preds, anchors, gt_boxes, gt_labels):
    M = gt_boxes.shape[0]
    far = jnp.float32(2.0e9)
    pad = jnp.full((MPAD - M, 4), far, gt_boxes.dtype)
    gt_pad = jnp.concatenate([gt_boxes, pad], axis=0)
    gt_t = jnp.zeros((8, MPAD), jnp.float32).at[0:4, :].set(gt_pad.T)
    lab_pad = jnp.concatenate(
        [gt_labels.astype(jnp.float32), jnp.zeros((MPAD - M,), jnp.float32)])
    tbl = jnp.zeros((MPAD, MPAD), jnp.float32)
    tbl = tbl.at[:, 0:4].set(gt_pad)
    tbl = tbl.at[:, 4].set(lab_pad)
    tbl = tbl.at[:, 5].set(1.0)
    bidx = jnp.arange(4 * N_BINS)
    gmask = (jnp.arange(MPAD)[None, :] == (bidx // N_BINS)[:, None]
             ).astype(jnp.float32)

    acx = (anchors[:, 0] + anchors[:, 2]) * 0.5
    acy = (anchors[:, 1] + anchors[:, 3]) * 0.5
    a6 = jnp.full((6, NP), 4.0e9, jnp.float32)
    a6 = a6.at[:, :N_ANCH].set(
        jnp.stack([acx, acy, anchors[:, 0], anchors[:, 1],
                   anchors[:, 2], anchors[:, 3]]).astype(jnp.float32))
    gcxp = (gt_pad[:, 0] + gt_pad[:, 2]) * 0.5
    gcyp = (gt_pad[:, 1] + gt_pad[:, 3]) * 0.5
    g6 = jnp.repeat(
        jnp.stack([gcxp, gcyp, gt_pad[:, 0], gt_pad[:, 1],
                   gt_pad[:, 2], gt_pad[:, 3]]).reshape(-1), 16)
    th_tile = _sc_thresh_call(a6.reshape(-1), g6)
    thresh = jnp.zeros((8, MPAD), jnp.float32).at[0].set(th_tile[:, 0])

    reg64 = reg_preds.reshape(N_ANCH, 4 * N_BINS)
    acc = _main_call(anchors, cls_preds, reg64, gt_t, tbl, gmask, thresh)
    bce_sum = jnp.sum(acc[0, :])
    reg_sum = jnp.sum(acc[1, :])
    npos = jnp.maximum(acc[2, 0], 1.0)
    return bce_sum / npos + reg_sum / (npos * 4.0)


# speedup vs baseline: 1.0077x; 1.0077x over previous
import functools

import jax
import jax.numpy as jnp
from jax import lax
from jax.experimental import pallas as pl
from jax.experimental.pallas import tpu as pltpu
from jax.experimental.pallas import tpu_sc as plsc

N_ANCH = 20000
N_CLS = 80
N_BINS = 16
TOP_K = 9
STRIDE = 8.0
MPAD = 128
BNM = 2000
NBLKM = N_ANCH // BNM
BIGF = 3.0e38
BIGI = 2**30


NP = 20480
GPT = 4


def _sc_thresh_kernel(cx_h, cy_h, x0_h, y0_h, x1_h, y1_h, g6_h, out_h,
                      cxv, cyv, gtv, idxv, av0, av1, av2, av3, bufv, sem):
    nc = 2
    wid = lax.axis_index("s") * nc + lax.axis_index("c")
    pltpu.sync_copy(cx_h, cxv)
    pltpu.sync_copy(cy_h, cyv)
    pltpu.sync_copy(g6_h, gtv)
    lio = lax.iota(jnp.int32, 16)
    lio9 = lio < TOP_K
    _dn = lax.GatherDimensionNumbers(
        offset_dims=(), collapsed_slice_dims=(0,), start_index_map=(0,))

    def _shuf(v, pidx):
        return lax.gather(v, pidx, _dn, (1,),
                          mode=lax.GatherScatterMode.PROMISE_IN_BOUNDS)

    _pidx = [(lio ^ k).reshape(16, 1) for k in (8, 4, 2, 1)]

    def _splat_sum(v):
        for p in _pidx:
            v = v + _shuf(v, p)
        return v

    for g in range(GPT):
        gid = wid * GPT + g

        def _gt(row, gid=gid):
            return gtv[pl.ds(row * (MPAD * 16) + gid * 16, 16)]

        gcx = _gt(0)
        gcy = _gt(1)
        gx0 = _gt(2)
        gy0 = _gt(3)
        gx1 = _gt(4)
        gy1 = _gt(5)

        def chunk_body(c, carry):
            ba = list(carry[0:TOP_K])
            ia = list(carry[TOP_K:2 * TOP_K])
            bb = list(carry[2 * TOP_K:3 * TOP_K])
            ib = list(carry[3 * TOP_K:])
            for u in range(2):
                base_a = c * 64 + u * 32
                base_b = base_a + 16
                cxa = cxv[pl.ds(base_a, 16)]
                cya = cyv[pl.ds(base_a, 16)]
                cxb = cxv[pl.ds(base_b, 16)]
                cyb = cyv[pl.ds(base_b, 16)]
                dxa = cxa - gcx
                dya = cya - gcy
                dxb = cxb - gcx
                dyb = cyb - gcy
                ta = dxa * dxa + dya * dya
                tb_ = dxb * dxb + dyb * dyb
                tia = lio + base_a
                tib = lio + base_b
                for j in range(TOP_K):
                    la = ta < ba[j]
                    lb = tb_ < bb[j]
                    nka = jnp.minimum(ta, ba[j])
                    nkb = jnp.minimum(tb_, bb[j])
                    xka = jnp.maximum(ta, ba[j])
                    xkb = jnp.maximum(tb_, bb[j])
                    nia = jnp.where(la, tia, ia[j])
                    nib = jnp.where(lb, tib, ib[j])
                    tia = jnp.where(la, ia[j], tia)
                    tib = jnp.where(lb, ib[j], tib)
                    ta = xka
                    tb_ = xkb
                    ba[j] = nka
                    ia[j] = nia
                    bb[j] = nkb
                    ib[j] = nib
            return tuple(ba) + tuple(ia) + tuple(bb) + tuple(ib)

        init = (tuple(jnp.full((16,), BIGF, jnp.float32)
                      for _ in range(TOP_K))
                + tuple(jnp.full((16,), BIGI, jnp.int32)
                        for _ in range(TOP_K))) * 2
        carry = lax.fori_loop(0, NP // 64, chunk_body, init)
        bk = carry[0:TOP_K] + carry[2 * TOP_K:3 * TOP_K]
        bi = carry[TOP_K:2 * TOP_K] + carry[3 * TOP_K:]

        last_k = jnp.full((16,), -1.0, jnp.float32)
        last_i = jnp.full((16,), -1, jnp.int32)
        selvec = jnp.zeros((16,), jnp.int32)
        for r in range(TOP_K):
            ck = jnp.full((16,), BIGF, jnp.float32)
            ci = jnp.full((16,), BIGI, jnp.int32)
            for j in range(2 * TOP_K):
                valid = (bk[j] > last_k) | ((bk[j] == last_k)
                                            & (bi[j] > last_i))
                kj = jnp.where(valid, bk[j], BIGF)
                ij = jnp.where(valid, bi[j], BIGI)
                lt = (kj < ck) | ((kj == ck) & (ij < ci))
                ck = jnp.where(lt, kj, ck)
                ci = jnp.where(lt, ij, ci)
            for p in _pidx:
                pk = _shuf(ck, p)
                pi = _shuf(ci, p)
                lt = (pk < ck) | ((pk == ck) & (pi < ci))
                ck = jnp.where(lt, pk, ck)
                ci = jnp.where(lt, pi, ci)
            last_k = ck
            last_i = ci
            selvec = jnp.where(lio == r, ci, selvec)

        idxv[...] = jnp.where(lio9, selvec, 0)
        h0 = pltpu.async_copy(x0_h.at[idxv], av0, sem)
        h1 = pltpu.async_copy(y0_h.at[idxv], av1, sem)
        h2 = pltpu.async_copy(x1_h.at[idxv], av2, sem)
        h3 = pltpu.async_copy(y1_h.at[idxv], av3, sem)
        h0.wait()
        h1.wait()
        h2.wait()
        h3.wait()
        ax0 = av0[...]
        ay0 = av1[...]
        ax1 = av2[...]
        ay1 = av3[...]
        area_a = (ax1 - ax0) * (ay1 - ay0)
        area_g = (gx1 - gx0) * (gy1 - gy0)
        iw = jnp.maximum(jnp.minimum(ax1, gx1) - jnp.maximum(ax0, gx0), 0.0)
        ih = jnp.maximum(jnp.minimum(ay1, gy1) - jnp.maximum(ay0, gy0), 0.0)
        inter = iw * ih
        union = area_a + area_g - inter
        iou = inter / jnp.maximum(union, 1e-9)
        s1 = _splat_sum(jnp.where(lio9, iou, 0.0))
        mean = s1 / jnp.float32(TOP_K)
        dev = iou - mean
        var = _splat_sum(jnp.where(lio9, dev * dev, 0.0)) \
            / jnp.float32(TOP_K - 1)
        seed = lax.bitcast_convert_type(
            (lax.bitcast_convert_type(var, jnp.int32) >> 1) + 0x1FBD1DF5,
            jnp.float32)
        y = seed
        for _ in range(3):
            y = 0.5 * (y + var / y)
        stdv = jnp.where(var > 0.0, y, 0.0)
        bufv[g, :] = mean + stdv

    pltpu.sync_copy(bufv, out_h.at[pl.ds(wid * GPT, GPT)])


def _sc_thresh_call(cxp, cyp, x0p, y0p, x1p, y1p, g6):
    mesh = plsc.VectorSubcoreMesh(core_axis_name="c", subcore_axis_name="s")
    fn = functools.partial(
        pl.kernel,
        out_type=jax.ShapeDtypeStruct((MPAD, 16), jnp.float32),
        mesh=mesh,
        scratch_types=[
            pltpu.VMEM((NP,), jnp.float32),
            pltpu.VMEM((NP,), jnp.float32),
            pltpu.VMEM((6 * MPAD * 16,), jnp.float32),
            pltpu.VMEM((16,), jnp.int32),
            pltpu.VMEM((16,), jnp.float32),
            pltpu.VMEM((16,), jnp.float32),
            pltpu.VMEM((16,), jnp.float32),
            pltpu.VMEM((16,), jnp.float32),
            pltpu.VMEM((GPT, 16), jnp.float32),
            pltpu.SemaphoreType.DMA,
        ],
    )(_sc_thresh_kernel)
    return fn(cxp, cyp, x0p, y0p, x1p, y1p, g6)


def _main_kernel(a_ref, cls_ref, reg_ref, gt_ref, tbl_ref, gmask_ref,
                 th_ref, out_ref):
    i = pl.program_id(0)
    ax0 = a_ref[:, 0:1]
    ay0 = a_ref[:, 1:2]
    ax1 = a_ref[:, 2:3]
    ay1 = a_ref[:, 3:4]
    gx0 = gt_ref[0:1, :]
    gy0 = gt_ref[1:2, :]
    gx1 = gt_ref[2:3, :]
    gy1 = gt_ref[3:4, :]
    acx = (ax0 + ax1) * 0.5
    acy = (ay0 + ay1) * 0.5
    area_a = (ax1 - ax0) * (ay1 - ay0)
    area_g = (gx1 - gx0) * (gy1 - gy0)
    iw = jnp.maximum(jnp.minimum(ax1, gx1) - jnp.maximum(ax0, gx0), 0.0)
    ih = jnp.maximum(jnp.minimum(ay1, gy1) - jnp.maximum(ay0, gy0), 0.0)
    inter = iw * ih
    iou = inter / jnp.maximum(area_a + area_g - inter, 1e-9)
    thresh = th_ref[0:1, :]
    inside = ((acx >= gx0) & (acx <= gx1) & (acy >= gy0) & (acy <= gy1))
    pos = (iou >= thresh) & inside
    lanem = jax.lax.broadcasted_iota(jnp.int32, (BNM, MPAD), 1)
    matched = jnp.max(jnp.where(pos, lanem, -1), axis=1, keepdims=True)
    sel = lanem == matched
    self_f = sel.astype(jnp.float32)
    gath = jnp.dot(self_f, tbl_ref[...],
                   preferred_element_type=jnp.float32)
    maxiou = jnp.dot(self_f * iou, tbl_ref[...],
                     preferred_element_type=jnp.float32)[:, 5:6]
    tbx0 = gath[:, 0:1]
    tby0 = gath[:, 1:2]
    tbx1 = gath[:, 2:3]
    tby1 = gath[:, 3:4]
    label = gath[:, 4:5]
    posf = gath[:, 5:6]
    q = maxiou * posf

    p = cls_ref[...]
    cio = jax.lax.broadcasted_iota(
        jnp.int32, (BNM, N_CLS), 1).astype(jnp.float32)
    t = jnp.where(cio == label, q, 0.0)
    bce = (jnp.maximum(p, 0.0) - p * t
           + jnp.log(1.0 + jnp.exp(-jnp.abs(p))))
    bce_row = jnp.sum(bce, axis=0, keepdims=True)

    tl = (acx - tbx0) / STRIDE
    tt = (acy - tby0) / STRIDE
    tr = (tbx1 - acx) / STRIDE
    tb = (tby1 - acy) / STRIDE
    r64 = reg_ref[...]
    bio = jax.lax.broadcasted_iota(jnp.int32, (BNM, 4 * N_BINS), 1)
    gid = bio // N_BINS
    binf = (bio % N_BINS).astype(jnp.float32)
    tgt64 = jnp.where(gid == 0, tl,
                      jnp.where(gid == 1, tt,
                                jnp.where(gid == 2, tr, tb)))
    tgt64 = jnp.clip(tgt64, 0.0, N_BINS - 1 - 1e-3)
    w = jnp.maximum(1.0 - jnp.abs(binf - tgt64), 0.0)
    ex = jnp.exp(r64)
    gm = gmask_ref[...]
    s4 = jnp.dot(ex, gm, preferred_element_type=jnp.float32)[:, 0:4]
    t4 = jnp.dot(r64 * w, gm, preferred_element_type=jnp.float32)[:, 0:4]
    elem4 = (jnp.log(s4) - t4) * posf
    reg_row = jnp.sum(elem4, axis=0, keepdims=True)
    npos_row = jnp.sum(posf, axis=0, keepdims=True)

    @pl.when(i == 0)
    def _():
        out_ref[...] = jnp.zeros(out_ref.shape, jnp.float32)

    out_ref[0:1, 0:N_CLS] += bce_row
    out_ref[1:2, 0:4] += reg_row
    out_ref[2:3, 0:1] += npos_row


def _main_call(anchors, cls_preds, reg64, gt_t, tbl, gmask, thresh):
    return pl.pallas_call(
        _main_kernel,
        grid=(NBLKM,),
        in_specs=[
            pl.BlockSpec((BNM, 4), lambda b: (b, 0)),
            pl.BlockSpec((BNM, N_CLS), lambda b: (b, 0)),
            pl.BlockSpec((BNM, 4 * N_BINS), lambda b: (b, 0)),
            pl.BlockSpec((8, MPAD), lambda b: (0, 0)),
            pl.BlockSpec((MPAD, MPAD), lambda b: (0, 0)),
            pl.BlockSpec((4 * N_BINS, MPAD), lambda b: (0, 0)),
            pl.BlockSpec((8, MPAD), lambda b: (0, 0)),
        ],
        out_specs=pl.BlockSpec((8, MPAD), lambda b: (0, 0)),
        out_shape=jax.ShapeDtypeStruct((8, MPAD), jnp.float32),
        compiler_params=pltpu.CompilerParams(
            dimension_semantics=("arbitrary",)),
    )(anchors, cls_preds, reg64, gt_t, tbl, gmask, thresh)


def kernel(cls_preds, reg_preds, anchors, gt_boxes, gt_labels):
    M = gt_boxes.shape[0]
    far = jnp.float32(2.0e9)
    pad = jnp.full((MPAD - M, 4), far, gt_boxes.dtype)
    gt_pad = jnp.concatenate([gt_boxes, pad], axis=0)
    gt_t = jnp.zeros((8, MPAD), jnp.float32).at[0:4, :].set(gt_pad.T)
    lab_pad = jnp.concatenate(
        [gt_labels.astype(jnp.float32), jnp.zeros((MPAD - M,), jnp.float32)])
    tbl = jnp.zeros((MPAD, MPAD), jnp.float32)
    tbl = tbl.at[:, 0:4].set(gt_pad)
    tbl = tbl.at[:, 4].set(lab_pad)
    tbl = tbl.at[:, 5].set(1.0)
    bidx = jnp.arange(4 * N_BINS)
    gmask = (jnp.arange(MPAD)[None, :] == (bidx // N_BINS)[:, None]
             ).astype(jnp.float32)

    acx = (anchors[:, 0] + anchors[:, 2]) * 0.5
    acy = (anchors[:, 1] + anchors[:, 3]) * 0.5
    padn = jnp.full((NP - N_ANCH,), 4.0e9, jnp.float32)

    def _padded(v):
        return jnp.concatenate([v.astype(jnp.float32), padn])

    cxp = _padded(acx)
    cyp = _padded(acy)
    x0p = _padded(anchors[:, 0])
    y0p = _padded(anchors[:, 1])
    x1p = _padded(anchors[:, 2])
    y1p = _padded(anchors[:, 3])
    gcxp = (gt_pad[:, 0] + gt_pad[:, 2]) * 0.5
    gcyp = (gt_pad[:, 1] + gt_pad[:, 3]) * 0.5
    g6 = jnp.repeat(
        jnp.stack([gcxp, gcyp, gt_pad[:, 0], gt_pad[:, 1],
                   gt_pad[:, 2], gt_pad[:, 3]]).reshape(-1), 16)
    th_tile = _sc_thresh_call(cxp, cyp, x0p, y0p, x1p, y1p, g6)
    thresh = jnp.zeros((8, MPAD), jnp.float32).at[0].set(th_tile[:, 0])

    reg64 = reg_preds.reshape(N_ANCH, 4 * N_BINS)
    acc = _main_call(anchors, cls_preds, reg64, gt_t, tbl, gmask, thresh)
    bce_sum = jnp.sum(acc[0, :])
    reg_sum = jnp.sum(acc[1, :])
    npos = jnp.maximum(acc[2, 0], 1.0)
    return bce_sum / npos + reg_sum / (npos * 4.0)
